# baseline (speedup 1.0000x reference)
import jax
import jax.numpy as jnp
from jax import lax
from jax.experimental import pallas as pl
from jax.experimental.pallas import tpu as pltpu
from jax.experimental.pallas import tpu_sc as plsc

NC = 2
NS = 16
NW = NC * NS

CHUNK = 16
N_STREAMS = 4
NBUF = 2


def _make_sc_call(B, S, L, D, V):
    SENT = B * S
    assert SENT % NW == 0
    sent_per_w = SENT // NW
    ipc = CHUNK * L
    ips = ipc // N_STREAMS
    assert ips <= 128 and ips % 8 == 0
    assert sent_per_w % (CHUNK * NBUF) == 0
    n_chunks = sent_per_w // CHUNK

    mesh = plsc.VectorSubcoreMesh(core_axis_name="c", subcore_axis_name="s")

    @pl.kernel(
        out_type=jax.ShapeDtypeStruct((SENT, D), jnp.float32),
        mesh=mesh,
        compiler_params=pltpu.CompilerParams(use_tc_tiling_on_sc=False,
                                             needs_layout_passes=False),
        scratch_types=[
            pltpu.VMEM((4, ipc), jnp.int32),
            pltpu.VMEM((4, ipc * 16), jnp.int32),
            pltpu.VMEM((NBUF * ipc, 2 * D), jnp.float32),
            pltpu.VMEM((CHUNK, D), jnp.float32),
            pltpu.VMEM((L, D), jnp.float32),
            pltpu.SemaphoreType.DMA,
            pltpu.SemaphoreType.DMA,
            pltpu.SemaphoreType.DMA,
            pltpu.SemaphoreType.DMA,
        ],
    )
    def sc_call(tpair_hbm, pid_hbm, off_hbm, mask_hbm, out_hbm,
                idx_v, off_v, rows_v, out_v, mask_v,
                sg0, sg1, si0, si1):
        wid = lax.axis_index("s") * NC + lax.axis_index("c")
        pltpu.sync_copy(mask_hbm, mask_v)
        sent_base = wid * sent_per_w
        idx_base = sent_base * L
        sems_g = [sg0, sg1]
        sems_i = [si0, si1]

        def stage_idx(c, islot, sem):
            src = pl.ds(idx_base + c * ipc, ipc)
            osrc = pl.ds((idx_base + c * ipc) * 16, ipc * 16)
            pltpu.async_copy(pid_hbm.at[src], idx_v.at[islot], sem)
            pltpu.async_copy(off_hbm.at[osrc], off_v.at[islot], sem)

        def wait_idx(c, islot, sem):
            src = pl.ds(idx_base + c * ipc, ipc)
            osrc = pl.ds((idx_base + c * ipc) * 16, ipc * 16)
            pltpu.make_async_copy(pid_hbm.at[src], idx_v.at[islot],
                                  sem).wait()
            pltpu.make_async_copy(off_hbm.at[osrc], off_v.at[islot],
                                  sem).wait()

        def fire_gathers(islot, b):
            for j in range(N_STREAMS):
                js = pl.ds(j * ips, ips)
                ds_dst = pl.ds(b * ipc + j * ips, ips)
                pltpu.async_copy(tpair_hbm.at[idx_v.at[islot, js]],
                                 rows_v.at[ds_dst], sems_g[b])

        def drain_gathers(islot, b):
            for j in range(N_STREAMS):
                js = pl.ds(j * ips, ips)
                ds_dst = pl.ds(b * ipc + j * ips, ips)
                pltpu.make_async_copy(tpair_hbm.at[idx_v.at[islot, js]],
                                      rows_v.at[ds_dst], sems_g[b]).wait()

        def compute(c, islot, b):
            masks = [[mask_v[l, pl.ds(dc * 16, 16)] for l in range(L)]
                     for dc in range(D // 16)]

            @pl.loop(0, CHUNK)
            def _sent(s):
                base = s * L
                accs = [None] * (D // 16)
                for l in range(L):
                    col0 = off_v[islot, pl.ds((base + l) * 16, 16)]
                    row16 = jnp.broadcast_to(b * ipc + base + l, (16,))
                    for dc in range(D // 16):
                        v = plsc.load_gather(rows_v,
                                             [row16, col0 + (dc * 16)])
                        t = v * masks[dc][l]
                        accs[dc] = t if accs[dc] is None else accs[dc] + t
                for dc in range(D // 16):
                    out_v[s, pl.ds(dc * 16, 16)] = accs[dc]

            pltpu.sync_copy(out_v,
                            out_hbm.at[pl.ds(sent_base + c * CHUNK, CHUNK)])

        for b in range(NBUF):
            stage_idx(b, b, sems_i[b])
            wait_idx(b, b, sems_i[b])
            fire_gathers(b, b)

        assert n_chunks % 4 == 0

        @pl.loop(0, n_chunks, step=4)
        def _chunks(c):
            for k in range(4):
                cc = c + k
                b = k % NBUF
                islot = k
                nslot = (k + 2) % 4
                nxt = cc + NBUF
                drain_gathers(islot, b)

                @pl.when(nxt < n_chunks)
                def _prefetch_idx():
                    stage_idx(nxt, nslot, sems_i[b])

                compute(cc, islot, b)

                @pl.when(nxt < n_chunks)
                def _fire_next():
                    wait_idx(nxt, nslot, sems_i[b])
                    fire_gathers(nslot, b)

    return sc_call


def kernel(stories, table, mask):
    B, S, L = stories.shape
    V, D = table.shape
    idx_flat = stories.astype(jnp.int32).reshape(-1)
    pair_id = idx_flat >> 1
    col0 = (((idx_flat & 1) * D)[:, None]
            + jnp.arange(16, dtype=jnp.int32)).reshape(-1)
    t_pair = table.reshape(V // 2, 2 * D)
    sc_call = _make_sc_call(B, S, L, D, V)
    out = sc_call(t_pair, pair_id, col0, mask.astype(jnp.float32))
    return out.reshape(B, S, D)
